# baseline (speedup 1.0000x reference)
import functools
import math

import jax
import jax.numpy as jnp
from jax import lax
from jax.experimental import pallas as pl
from jax.experimental.pallas import tpu as pltpu
from jax.experimental.pallas import tpu_sc as plsc

E = 8
K = 2
H = 1024
F = 4096

BLK = 256
JBLK = 512
NJ = F // JBLK

_SQRT2 = math.sqrt(2.0)


def _gelu(h):
    return 0.5 * h * (1.0 + jax.lax.erf(h / _SQRT2))


def _gmm_body(es_ref, xs_ref, cw_ref, w1_ref, b1_ref, w2_ref, b2_ref,
              ys_ref, acc_ref, xb_ref):
    j = pl.program_id(1)

    @pl.when(j == 0)
    def _():
        acc_ref[...] = jnp.zeros_like(acc_ref)
        xb_ref[...] = xs_ref[...].astype(jnp.bfloat16)

    h = jnp.dot(xb_ref[...], w1_ref[0], preferred_element_type=jnp.float32)
    h = _gelu(h + b1_ref[0]).astype(jnp.bfloat16)
    acc_ref[...] += jnp.dot(h, w2_ref[0], preferred_element_type=jnp.float32)

    @pl.when(j == NJ - 1)
    def _():
        ys_ref[...] = (acc_ref[...] + b2_ref[0]) * cw_ref[:, :1]


def _gmm(block_expert, xs, cw, w1, b1, w2, b2, nb):
    grid_spec = pltpu.PrefetchScalarGridSpec(
        num_scalar_prefetch=1,
        grid=(nb, NJ),
        in_specs=[
            pl.BlockSpec((BLK, H), lambda r, j, es: (r, 0)),
            pl.BlockSpec((BLK, 128), lambda r, j, es: (r, 0)),
            pl.BlockSpec((1, H, JBLK), lambda r, j, es: (es[r], 0, j)),
            pl.BlockSpec((1, 1, JBLK), lambda r, j, es: (es[r], 0, j)),
            pl.BlockSpec((1, JBLK, H), lambda r, j, es: (es[r], j, 0)),
            pl.BlockSpec((1, 1, H), lambda r, j, es: (es[r], 0, 0)),
        ],
        out_specs=pl.BlockSpec((BLK, H), lambda r, j, es: (r, 0)),
        scratch_shapes=[pltpu.VMEM((BLK, H), jnp.float32),
                        pltpu.VMEM((BLK, H), jnp.bfloat16)],
    )
    return pl.pallas_call(
        _gmm_body,
        grid_spec=grid_spec,
        out_shape=jax.ShapeDtypeStruct((nb * BLK, H), jnp.float32),
        compiler_params=pltpu.CompilerParams(
            dimension_semantics=("arbitrary", "arbitrary")),
    )(block_expert, xs, cw,
      w1.astype(jnp.bfloat16), b1.reshape(E, 1, F),
      w2.astype(jnp.bfloat16), b2.reshape(E, 1, H))


_NC = 2
_NS = 16
_NW = _NC * _NS


def _sc_gather(sorted_tok, xf, padded, t_max):
    rows_w = padded // _NW
    ch = 16
    nch = rows_w // ch
    nbuf = 4
    look = 2
    mesh = plsc.VectorSubcoreMesh(core_axis_name="c", subcore_axis_name="s")

    @functools.partial(
        pl.kernel,
        out_type=jax.ShapeDtypeStruct((padded, H), jnp.float32),
        mesh=mesh,
        scratch_types=[
            pltpu.VMEM((rows_w,), jnp.int32),
            *[pltpu.VMEM((ch, H), jnp.float32) for _ in range(nbuf)],
            *[pltpu.SemaphoreType.DMA for _ in range(2 * nbuf)],
        ],
    )
    def k(tok_hbm, x_hbm, xs_hbm, idx_v, *bufs_sems):
        bufs = bufs_sems[:nbuf]
        gsem = bufs_sems[nbuf:2 * nbuf]
        osem = bufs_sems[2 * nbuf:]
        wid = lax.axis_index("s") * _NC + lax.axis_index("c")
        base = wid * rows_w
        pltpu.sync_copy(tok_hbm.at[pl.ds(base, rows_w)], idx_v)
        for i in range(rows_w // 16):
            v = idx_v[pl.ds(i * 16, 16)]
            idx_v[pl.ds(i * 16, 16)] = jnp.clip(v, 0, t_max - 1)

        def start_in(c):
            cp = pltpu.make_async_copy(
                x_hbm.at[idx_v.at[pl.ds(c * ch, ch)]],
                bufs[c % nbuf], gsem[c % nbuf])
            cp.start()
            return cp

        def start_out(c):
            cp = pltpu.make_async_copy(
                bufs[c % nbuf], xs_hbm.at[pl.ds(base + c * ch, ch)],
                osem[c % nbuf])
            cp.start()
            return cp

        in_cp = {}
        out_cp = {}
        for c in range(min(look, nch)):
            in_cp[c] = start_in(c)
        for c in range(nch):
            nxt = c + look
            if nxt < nch:
                if nxt >= nbuf:
                    out_cp[nxt - nbuf].wait()
                in_cp[nxt] = start_in(nxt)
            in_cp[c].wait()
            out_cp[c] = start_out(c)
        for c in range(max(0, nch - nbuf), nch):
            out_cp[c].wait()

    return k(sorted_tok, xf)


def _sc_combine(ys, pos0, pos1, t_tokens):
    rows_w = t_tokens // _NW
    ch = 16
    nch = rows_w // ch
    mesh = plsc.VectorSubcoreMesh(core_axis_name="c", subcore_axis_name="s")

    @functools.partial(
        pl.kernel,
        out_type=jax.ShapeDtypeStruct((t_tokens, H), jnp.float32),
        mesh=mesh,
        scratch_types=[
            pltpu.VMEM((rows_w,), jnp.int32),
            pltpu.VMEM((rows_w,), jnp.int32),
            pltpu.VMEM((ch, H), jnp.float32),
            pltpu.VMEM((ch, H), jnp.float32),
            pltpu.SemaphoreType.DMA,
            pltpu.SemaphoreType.DMA,
        ],
    )
    def k(p0_hbm, p1_hbm, ys_hbm, out_hbm, i0_v, i1_v, bufa, bufb, sema, semb):
        wid = lax.axis_index("s") * _NC + lax.axis_index("c")
        base = wid * rows_w
        pltpu.sync_copy(p0_hbm.at[pl.ds(base, rows_w)], i0_v)
        pltpu.sync_copy(p1_hbm.at[pl.ds(base, rows_w)], i1_v)
        for c in range(nch):
            cpa = pltpu.make_async_copy(
                ys_hbm.at[i0_v.at[pl.ds(c * ch, ch)]], bufa, sema)
            cpb = pltpu.make_async_copy(
                ys_hbm.at[i1_v.at[pl.ds(c * ch, ch)]], bufb, semb)
            cpa.start()
            cpb.start()
            cpa.wait()
            cpb.wait()

            def body(r, _):
                for o in range(H // 16):
                    s = pl.ds(o * 16, 16)
                    bufa[r, s] = bufa[r, s] + bufb[r, s]
                return 0

            lax.fori_loop(0, ch, body, 0)
            pltpu.sync_copy(bufa, out_hbm.at[pl.ds(base + c * ch, ch)])

    return k(pos0, pos1, ys)


def kernel(x, gate_w, gate_b, w1, b1, w2, b2):
    B, S, _ = x.shape
    T = B * S
    A = K * T
    nb = A // BLK + E
    padded = nb * BLK

    xf = x.reshape(T, H)

    logits = xf @ gate_w + gate_b
    w = jax.nn.softmax(logits, axis=-1)
    tkw, tki = jax.lax.top_k(w, K)
    tkw = tkw / jnp.clip(jnp.sum(tkw, axis=-1, keepdims=True), 1e-6, None)

    e_flat = tki.reshape(-1).astype(jnp.int32)
    w_flat = tkw.reshape(-1)
    tok_flat = (jnp.arange(A, dtype=jnp.int32) // K)

    onehot = (e_flat[:, None] == jnp.arange(E, dtype=jnp.int32)[None, :])
    counts = jnp.sum(onehot.astype(jnp.int32), axis=0)
    seg_len = ((counts + BLK - 1) // BLK) * BLK
    starts = jnp.concatenate([jnp.zeros((1,), jnp.int32),
                              jnp.cumsum(seg_len)[:-1].astype(jnp.int32)])
    rank = jnp.cumsum(onehot.astype(jnp.int32), axis=0) - 1
    rank_own = jnp.take_along_axis(rank, e_flat[:, None], axis=1)[:, 0]
    pos = starts[e_flat] + rank_own

    ends = starts + seg_len
    block_start = jnp.arange(nb, dtype=jnp.int32) * BLK
    block_expert = jnp.minimum(
        jnp.sum((block_start[:, None] >= ends[None, :]).astype(jnp.int32),
                axis=1), E - 1).astype(jnp.int32)

    sorted_tok = jnp.zeros((padded,), jnp.int32).at[pos].set(tok_flat)
    sorted_cw = jnp.zeros((padded,), jnp.float32).at[pos].set(w_flat)
    xs = _sc_gather(sorted_tok, xf, padded, T)
    cw2 = jnp.broadcast_to(sorted_cw[:, None], (padded, 128))

    ys = _gmm(block_expert, xs, cw2, w1, b1, w2, b2, nb)

    pos2 = pos.reshape(T, K)
    out = _sc_combine(ys, pos2[:, 0], pos2[:, 1], T)
    return out.reshape(B, S, H)


# speedup vs baseline: 1.3019x; 1.3019x over previous
import functools
import math

import jax
import jax.numpy as jnp
from jax import lax
from jax.experimental import pallas as pl
from jax.experimental.pallas import tpu as pltpu
from jax.experimental.pallas import tpu_sc as plsc

E = 8
K = 2
H = 1024
F = 4096

BLK = 512
JBLK = 1024
NJ = F // JBLK

_SQRT2 = math.sqrt(2.0)


def _gelu(h):
    return 0.5 * h * (1.0 + jax.lax.erf(h / _SQRT2))


def _gmm_body(es_ref, xs_ref, cw_ref, w1_ref, b1_ref, w2_ref, b2_ref,
              ys_ref, acc_ref):
    r = pl.program_id(0)
    j = pl.program_id(1)

    @pl.when(es_ref[r] < E)
    def _():
        @pl.when(j == 0)
        def _():
            acc_ref[...] = jnp.zeros_like(acc_ref)

        h = jnp.dot(xs_ref[...], w1_ref[0],
                    preferred_element_type=jnp.float32)
        h = _gelu(h + b1_ref[0])
        acc_ref[...] += jnp.dot(h, w2_ref[0],
                                preferred_element_type=jnp.float32)

        @pl.when(j == NJ - 1)
        def _():
            ys_ref[...] = (acc_ref[...] + b2_ref[0]) * cw_ref[:, :1]


def _gmm(block_expert, xs, cw, w1, b1, w2, b2, nb):
    grid_spec = pltpu.PrefetchScalarGridSpec(
        num_scalar_prefetch=1,
        grid=(nb, NJ),
        in_specs=[
            pl.BlockSpec((BLK, H), lambda r, j, es: (r, 0)),
            pl.BlockSpec((BLK, 128), lambda r, j, es: (r, 0)),
            pl.BlockSpec((1, H, JBLK),
                         lambda r, j, es: (jnp.minimum(es[r], E - 1), 0, j)),
            pl.BlockSpec((1, 1, JBLK),
                         lambda r, j, es: (jnp.minimum(es[r], E - 1), 0, j)),
            pl.BlockSpec((1, JBLK, H),
                         lambda r, j, es: (jnp.minimum(es[r], E - 1), j, 0)),
            pl.BlockSpec((1, 1, H),
                         lambda r, j, es: (jnp.minimum(es[r], E - 1), 0, 0)),
        ],
        out_specs=pl.BlockSpec((BLK, H), lambda r, j, es: (r, 0)),
        scratch_shapes=[pltpu.VMEM((BLK, H), jnp.float32)],
    )
    return pl.pallas_call(
        _gmm_body,
        grid_spec=grid_spec,
        out_shape=jax.ShapeDtypeStruct((nb * BLK, H), jnp.float32),
        compiler_params=pltpu.CompilerParams(
            dimension_semantics=("arbitrary", "arbitrary")),
    )(block_expert, xs, cw,
      w1, b1.reshape(E, 1, F), w2, b2.reshape(E, 1, H))


_NC = 2
_NS = 16
_NW = _NC * _NS


def _sc_gather(sorted_tok, xf, padded, t_max):
    rows_w = padded // _NW
    ch = 16
    nch = rows_w // ch
    nbuf = 4
    look = 2
    mesh = plsc.VectorSubcoreMesh(core_axis_name="c", subcore_axis_name="s")

    @functools.partial(
        pl.kernel,
        out_type=jax.ShapeDtypeStruct((padded, H), jnp.float32),
        mesh=mesh,
        scratch_types=[
            pltpu.VMEM((rows_w,), jnp.int32),
            *[pltpu.VMEM((ch, H), jnp.float32) for _ in range(nbuf)],
            *[pltpu.SemaphoreType.DMA for _ in range(2 * nbuf)],
        ],
    )
    def k(tok_hbm, x_hbm, xs_hbm, idx_v, *bufs_sems):
        bufs = bufs_sems[:nbuf]
        gsem = bufs_sems[nbuf:2 * nbuf]
        osem = bufs_sems[2 * nbuf:]
        wid = lax.axis_index("s") * _NC + lax.axis_index("c")
        base = wid * rows_w
        pltpu.sync_copy(tok_hbm.at[pl.ds(base, rows_w)], idx_v)
        for i in range(rows_w // 16):
            v = idx_v[pl.ds(i * 16, 16)]
            idx_v[pl.ds(i * 16, 16)] = jnp.clip(v, 0, t_max - 1)

        def start_in(c):
            cp = pltpu.make_async_copy(
                x_hbm.at[idx_v.at[pl.ds(c * ch, ch)]],
                bufs[c % nbuf], gsem[c % nbuf])
            cp.start()
            return cp

        def start_out(c):
            cp = pltpu.make_async_copy(
                bufs[c % nbuf], xs_hbm.at[pl.ds(base + c * ch, ch)],
                osem[c % nbuf])
            cp.start()
            return cp

        in_cp = {}
        out_cp = {}
        for c in range(min(look, nch)):
            in_cp[c] = start_in(c)
        for c in range(nch):
            nxt = c + look
            if nxt < nch:
                if nxt >= nbuf:
                    out_cp[nxt - nbuf].wait()
                in_cp[nxt] = start_in(nxt)
            in_cp[c].wait()
            out_cp[c] = start_out(c)
        for c in range(max(0, nch - nbuf), nch):
            out_cp[c].wait()

    return k(sorted_tok, xf)


def _sc_combine(ys, pos0, pos1, t_tokens):
    rows_w = t_tokens // _NW
    ch = 16
    nch = rows_w // ch
    mesh = plsc.VectorSubcoreMesh(core_axis_name="c", subcore_axis_name="s")

    @functools.partial(
        pl.kernel,
        out_type=jax.ShapeDtypeStruct((t_tokens, H), jnp.float32),
        mesh=mesh,
        scratch_types=[
            pltpu.VMEM((rows_w,), jnp.int32),
            pltpu.VMEM((rows_w,), jnp.int32),
            pltpu.VMEM((ch, H), jnp.float32),
            pltpu.VMEM((ch, H), jnp.float32),
            pltpu.SemaphoreType.DMA,
            pltpu.SemaphoreType.DMA,
        ],
    )
    def k(p0_hbm, p1_hbm, ys_hbm, out_hbm, i0_v, i1_v, bufa, bufb, sema, semb):
        wid = lax.axis_index("s") * _NC + lax.axis_index("c")
        base = wid * rows_w
        pltpu.sync_copy(p0_hbm.at[pl.ds(base, rows_w)], i0_v)
        pltpu.sync_copy(p1_hbm.at[pl.ds(base, rows_w)], i1_v)
        for c in range(nch):
            cpa = pltpu.make_async_copy(
                ys_hbm.at[i0_v.at[pl.ds(c * ch, ch)]], bufa, sema)
            cpb = pltpu.make_async_copy(
                ys_hbm.at[i1_v.at[pl.ds(c * ch, ch)]], bufb, semb)
            cpa.start()
            cpb.start()
            cpa.wait()
            cpb.wait()

            def body(r, _):
                for o in range(H // 16):
                    s = pl.ds(o * 16, 16)
                    bufa[r, s] = bufa[r, s] + bufb[r, s]
                return 0

            lax.fori_loop(0, ch, body, 0)
            pltpu.sync_copy(bufa, out_hbm.at[pl.ds(base + c * ch, ch)])

    return k(pos0, pos1, ys)


def kernel(x, gate_w, gate_b, w1, b1, w2, b2):
    B, S, _ = x.shape
    T = B * S
    A = K * T
    nb = A // BLK + E
    padded = nb * BLK

    xf = x.reshape(T, H)

    logits = xf @ gate_w + gate_b
    w = jax.nn.softmax(logits, axis=-1)
    tkw, tki = jax.lax.top_k(w, K)
    tkw = tkw / jnp.clip(jnp.sum(tkw, axis=-1, keepdims=True), 1e-6, None)

    e_flat = tki.reshape(-1).astype(jnp.int32)
    w_flat = tkw.reshape(-1)
    tok_flat = (jnp.arange(A, dtype=jnp.int32) // K)

    onehot = (e_flat[:, None] == jnp.arange(E, dtype=jnp.int32)[None, :])
    counts = jnp.sum(onehot.astype(jnp.int32), axis=0)
    seg_len = ((counts + BLK - 1) // BLK) * BLK
    starts = jnp.concatenate([jnp.zeros((1,), jnp.int32),
                              jnp.cumsum(seg_len)[:-1].astype(jnp.int32)])
    rank = jnp.cumsum(onehot.astype(jnp.int32), axis=0) - 1
    rank_own = jnp.take_along_axis(rank, e_flat[:, None], axis=1)[:, 0]
    pos = starts[e_flat] + rank_own

    ends = starts + seg_len
    block_start = jnp.arange(nb, dtype=jnp.int32) * BLK
    block_expert = jnp.sum(
        (block_start[:, None] >= ends[None, :]).astype(jnp.int32),
        axis=1).astype(jnp.int32)

    sorted_tok = jnp.zeros((padded,), jnp.int32).at[pos].set(tok_flat)
    sorted_cw = jnp.zeros((padded,), jnp.float32).at[pos].set(w_flat)
    xs = _sc_gather(sorted_tok, xf, padded, T)
    cw2 = jnp.broadcast_to(sorted_cw[:, None], (padded, 128))

    ys = _gmm(block_expert, xs, cw2, w1, b1, w2, b2, nb)

    pos2 = pos.reshape(T, K)
    out = _sc_combine(ys, pos2[:, 0], pos2[:, 1], T)
    return out.reshape(B, S, H)


# speedup vs baseline: 1.3270x; 1.0193x over previous
import functools
import math

import jax
import jax.numpy as jnp
from jax import lax
from jax.experimental import pallas as pl
from jax.experimental.pallas import tpu as pltpu
from jax.experimental.pallas import tpu_sc as plsc

E = 8
K = 2
H = 1024
F = 4096

BLK = 512
JBLK = 1024
NJ = F // JBLK

_SQRT2 = math.sqrt(2.0)


def _gelu(h):
    return 0.5 * h * (1.0 + jax.lax.erf(h / _SQRT2))


def _gmm_body(es_ref, xs_ref, cw_ref, w1_ref, b1_ref, w2_ref, b2_ref,
              ys_ref, acc_ref):
    r = pl.program_id(0)
    j = pl.program_id(1)

    @pl.when(es_ref[r] < E)
    def _():
        @pl.when(j == 0)
        def _():
            acc_ref[...] = jnp.zeros_like(acc_ref)

        h = jnp.dot(xs_ref[...], w1_ref[0],
                    preferred_element_type=jnp.float32)
        h = _gelu(h + b1_ref[0])
        acc_ref[...] += jnp.dot(h, w2_ref[0],
                                preferred_element_type=jnp.float32)

        @pl.when(j == NJ - 1)
        def _():
            ys_ref[...] = (acc_ref[...] + b2_ref[0]) * cw_ref[:, :1]


def _gmm(block_expert, xs, cw, w1, b1, w2, b2, nb):
    grid_spec = pltpu.PrefetchScalarGridSpec(
        num_scalar_prefetch=1,
        grid=(nb, NJ),
        in_specs=[
            pl.BlockSpec((BLK, H), lambda r, j, es: (r, 0)),
            pl.BlockSpec((BLK, 128), lambda r, j, es: (r, 0)),
            pl.BlockSpec((1, H, JBLK),
                         lambda r, j, es: (jnp.minimum(es[r], E - 1), 0, j)),
            pl.BlockSpec((1, 1, JBLK),
                         lambda r, j, es: (jnp.minimum(es[r], E - 1), 0, j)),
            pl.BlockSpec((1, JBLK, H),
                         lambda r, j, es: (jnp.minimum(es[r], E - 1), j, 0)),
            pl.BlockSpec((1, 1, H),
                         lambda r, j, es: (jnp.minimum(es[r], E - 1), 0, 0)),
        ],
        out_specs=pl.BlockSpec((BLK, H), lambda r, j, es: (r, 0)),
        scratch_shapes=[pltpu.VMEM((BLK, H), jnp.float32)],
    )
    return pl.pallas_call(
        _gmm_body,
        grid_spec=grid_spec,
        out_shape=jax.ShapeDtypeStruct((nb * BLK, H), jnp.float32),
        compiler_params=pltpu.CompilerParams(
            dimension_semantics=("arbitrary", "arbitrary")),
    )(block_expert, xs, cw,
      w1, b1.reshape(E, 1, F), w2, b2.reshape(E, 1, H))


_NC = 2
_NS = 16
_NW = _NC * _NS


def _sc_gather(sorted_tok, xf, padded, t_max):
    rows_w = padded // _NW
    ch = 16
    nch = rows_w // ch
    nbuf = 4
    look = 2
    mesh = plsc.VectorSubcoreMesh(core_axis_name="c", subcore_axis_name="s")

    @functools.partial(
        pl.kernel,
        out_type=jax.ShapeDtypeStruct((padded, H), jnp.float32),
        mesh=mesh,
        scratch_types=[
            pltpu.VMEM((rows_w,), jnp.int32),
            *[pltpu.VMEM((ch, H), jnp.float32) for _ in range(nbuf)],
            *[pltpu.SemaphoreType.DMA for _ in range(2 * nbuf)],
        ],
    )
    def k(tok_hbm, x_hbm, xs_hbm, idx_v, *bufs_sems):
        bufs = bufs_sems[:nbuf]
        gsem = bufs_sems[nbuf:2 * nbuf]
        osem = bufs_sems[2 * nbuf:]
        wid = lax.axis_index("s") * _NC + lax.axis_index("c")
        base = wid * rows_w
        pltpu.sync_copy(tok_hbm.at[pl.ds(base, rows_w)], idx_v)
        for i in range(rows_w // 16):
            v = idx_v[pl.ds(i * 16, 16)]
            idx_v[pl.ds(i * 16, 16)] = jnp.clip(v, 0, t_max - 1)

        def start_in(c):
            cp = pltpu.make_async_copy(
                x_hbm.at[idx_v.at[pl.ds(c * ch, ch)]],
                bufs[c % nbuf], gsem[c % nbuf])
            cp.start()
            return cp

        def start_out(c):
            cp = pltpu.make_async_copy(
                bufs[c % nbuf], xs_hbm.at[pl.ds(base + c * ch, ch)],
                osem[c % nbuf])
            cp.start()
            return cp

        in_cp = {}
        out_cp = {}
        for c in range(min(look, nch)):
            in_cp[c] = start_in(c)
        for c in range(nch):
            nxt = c + look
            if nxt < nch:
                if nxt >= nbuf:
                    out_cp[nxt - nbuf].wait()
                in_cp[nxt] = start_in(nxt)
            in_cp[c].wait()
            out_cp[c] = start_out(c)
        for c in range(max(0, nch - nbuf), nch):
            out_cp[c].wait()

    return k(sorted_tok, xf)


def _sc_combine(ys, pos0, pos1, t_tokens):
    rows_w = t_tokens // _NW
    ch = 16
    nch = rows_w // ch
    mesh = plsc.VectorSubcoreMesh(core_axis_name="c", subcore_axis_name="s")

    nslot = 2
    @functools.partial(
        pl.kernel,
        out_type=jax.ShapeDtypeStruct((t_tokens, H), jnp.float32),
        mesh=mesh,
        scratch_types=[
            pltpu.VMEM((rows_w,), jnp.int32),
            pltpu.VMEM((rows_w,), jnp.int32),
            *[pltpu.VMEM((ch, H), jnp.float32) for _ in range(2 * nslot)],
            *[pltpu.SemaphoreType.DMA for _ in range(3 * nslot)],
        ],
    )
    def k(p0_hbm, p1_hbm, ys_hbm, out_hbm, i0_v, i1_v, *bufs_sems):
        bufa = bufs_sems[:nslot]
        bufb = bufs_sems[nslot:2 * nslot]
        sema = bufs_sems[2 * nslot:3 * nslot]
        semb = bufs_sems[3 * nslot:4 * nslot]
        semo = bufs_sems[4 * nslot:]
        wid = lax.axis_index("s") * _NC + lax.axis_index("c")
        base = wid * rows_w
        pltpu.sync_copy(p0_hbm.at[pl.ds(base, rows_w)], i0_v)
        pltpu.sync_copy(p1_hbm.at[pl.ds(base, rows_w)], i1_v)

        def start_in(c):
            s = c % nslot
            cpa = pltpu.make_async_copy(
                ys_hbm.at[i0_v.at[pl.ds(c * ch, ch)]], bufa[s], sema[s])
            cpb = pltpu.make_async_copy(
                ys_hbm.at[i1_v.at[pl.ds(c * ch, ch)]], bufb[s], semb[s])
            cpa.start()
            cpb.start()
            return cpa, cpb

        in_cp = {}
        out_cp = {}
        for c in range(min(1, nch)):
            in_cp[c] = start_in(c)
        for c in range(nch):
            s = c % nslot
            nxt = c + 1
            if nxt < nch:
                if nxt >= nslot:
                    out_cp[nxt - nslot].wait()
                in_cp[nxt] = start_in(nxt)
            in_cp[c][0].wait()
            in_cp[c][1].wait()

            def body(r, _):
                for o in range(H // 16):
                    sl = pl.ds(o * 16, 16)
                    bufa[s][r, sl] = bufa[s][r, sl] + bufb[s][r, sl]
                return 0

            lax.fori_loop(0, ch, body, 0)
            cpo = pltpu.make_async_copy(
                bufa[s], out_hbm.at[pl.ds(base + c * ch, ch)], semo[s])
            cpo.start()
            out_cp[c] = cpo
        for c in range(max(0, nch - nslot), nch):
            out_cp[c].wait()

    return k(pos0, pos1, ys)


def kernel(x, gate_w, gate_b, w1, b1, w2, b2):
    B, S, _ = x.shape
    T = B * S
    A = K * T
    nb = A // BLK + E
    padded = nb * BLK

    xf = x.reshape(T, H)

    logits = xf @ gate_w + gate_b
    w = jax.nn.softmax(logits, axis=-1)
    tkw, tki = jax.lax.top_k(w, K)
    tkw = tkw / jnp.clip(jnp.sum(tkw, axis=-1, keepdims=True), 1e-6, None)

    e_flat = tki.reshape(-1).astype(jnp.int32)
    w_flat = tkw.reshape(-1)
    tok_flat = (jnp.arange(A, dtype=jnp.int32) // K)

    onehot = (e_flat[:, None] == jnp.arange(E, dtype=jnp.int32)[None, :])
    counts = jnp.sum(onehot.astype(jnp.int32), axis=0)
    seg_len = ((counts + BLK - 1) // BLK) * BLK
    starts = jnp.concatenate([jnp.zeros((1,), jnp.int32),
                              jnp.cumsum(seg_len)[:-1].astype(jnp.int32)])
    rank = jnp.cumsum(onehot.astype(jnp.int32), axis=0) - 1
    rank_own = jnp.take_along_axis(rank, e_flat[:, None], axis=1)[:, 0]
    pos = starts[e_flat] + rank_own

    ends = starts + seg_len
    block_start = jnp.arange(nb, dtype=jnp.int32) * BLK
    block_expert = jnp.sum(
        (block_start[:, None] >= ends[None, :]).astype(jnp.int32),
        axis=1).astype(jnp.int32)

    sorted_tok = jnp.zeros((padded,), jnp.int32).at[pos].set(tok_flat)
    sorted_cw = jnp.zeros((padded,), jnp.float32).at[pos].set(w_flat)
    xs = _sc_gather(sorted_tok, xf, padded, T)
    cw2 = jnp.broadcast_to(sorted_cw[:, None], (padded, 128))

    ys = _gmm(block_expert, xs, cw2, w1, b1, w2, b2, nb)

    pos2 = pos.reshape(T, K)
    out = _sc_combine(ys, pos2[:, 0], pos2[:, 1], T)
    return out.reshape(B, S, H)


# speedup vs baseline: 1.3304x; 1.0026x over previous
import functools
import math

import jax
import jax.numpy as jnp
from jax import lax
from jax.experimental import pallas as pl
from jax.experimental.pallas import tpu as pltpu
from jax.experimental.pallas import tpu_sc as plsc

E = 8
K = 2
H = 1024
F = 4096

BLK = 512
JBLK = 1024
NJ = F // JBLK

_SQRT2 = math.sqrt(2.0)


def _gelu(h):
    return 0.5 * h * (1.0 + jax.lax.erf(h / _SQRT2))


def _gmm_body(es_ref, xs_ref, cw_ref, w1_ref, b1_ref, w2_ref, b2_ref,
              ys_ref, acc_ref):
    r = pl.program_id(0)
    j = pl.program_id(1)

    @pl.when(es_ref[r] < E)
    def _():
        @pl.when(j == 0)
        def _():
            acc_ref[...] = jnp.zeros_like(acc_ref)

        h = jnp.dot(xs_ref[...], w1_ref[0],
                    preferred_element_type=jnp.float32)
        h = _gelu(h + b1_ref[0])
        acc_ref[...] += jnp.dot(h, w2_ref[0],
                                preferred_element_type=jnp.float32)

        @pl.when(j == NJ - 1)
        def _():
            ys_ref[...] = (acc_ref[...] + b2_ref[0]) * cw_ref[:, :1]


def _gmm(block_expert, xs, cw, w1, b1, w2, b2, nb):
    grid_spec = pltpu.PrefetchScalarGridSpec(
        num_scalar_prefetch=1,
        grid=(nb, NJ),
        in_specs=[
            pl.BlockSpec((BLK, H), lambda r, j, es: (r, 0)),
            pl.BlockSpec((BLK, 128), lambda r, j, es: (r, 0)),
            pl.BlockSpec((1, H, JBLK),
                         lambda r, j, es: (jnp.minimum(es[r], E - 1), 0, j)),
            pl.BlockSpec((1, 1, JBLK),
                         lambda r, j, es: (jnp.minimum(es[r], E - 1), 0, j)),
            pl.BlockSpec((1, JBLK, H),
                         lambda r, j, es: (jnp.minimum(es[r], E - 1), j, 0)),
            pl.BlockSpec((1, 1, H),
                         lambda r, j, es: (jnp.minimum(es[r], E - 1), 0, 0)),
        ],
        out_specs=pl.BlockSpec((BLK, H), lambda r, j, es: (r, 0)),
        scratch_shapes=[pltpu.VMEM((BLK, H), jnp.float32)],
    )
    return pl.pallas_call(
        _gmm_body,
        grid_spec=grid_spec,
        out_shape=jax.ShapeDtypeStruct((nb * BLK, H), jnp.float32),
        compiler_params=pltpu.CompilerParams(
            dimension_semantics=("arbitrary", "arbitrary")),
    )(block_expert, xs, cw,
      w1, b1.reshape(E, 1, F), w2, b2.reshape(E, 1, H))


_NC = 2
_NS = 16
_NW = _NC * _NS


def _sc_gather(sorted_tok, xf, padded, t_max):
    rows_w = padded // _NW
    ch = 16
    nch = rows_w // ch
    nbuf = 4
    look = 2
    mesh = plsc.VectorSubcoreMesh(core_axis_name="c", subcore_axis_name="s")

    @functools.partial(
        pl.kernel,
        out_type=jax.ShapeDtypeStruct((padded, H), jnp.float32),
        mesh=mesh,
        scratch_types=[
            pltpu.VMEM((rows_w,), jnp.int32),
            *[pltpu.VMEM((ch, H), jnp.float32) for _ in range(nbuf)],
            *[pltpu.SemaphoreType.DMA for _ in range(2 * nbuf)],
        ],
    )
    def moe_sc_gather(tok_hbm, x_hbm, xs_hbm, idx_v, *bufs_sems):
        bufs = bufs_sems[:nbuf]
        gsem = bufs_sems[nbuf:2 * nbuf]
        osem = bufs_sems[2 * nbuf:]
        wid = lax.axis_index("s") * _NC + lax.axis_index("c")
        base = wid * rows_w
        pltpu.sync_copy(tok_hbm.at[pl.ds(base, rows_w)], idx_v)
        for i in range(rows_w // 16):
            v = idx_v[pl.ds(i * 16, 16)]
            idx_v[pl.ds(i * 16, 16)] = jnp.clip(v, 0, t_max - 1)

        def start_in(c):
            cp = pltpu.make_async_copy(
                x_hbm.at[idx_v.at[pl.ds(c * ch, ch)]],
                bufs[c % nbuf], gsem[c % nbuf])
            cp.start()
            return cp

        def start_out(c):
            cp = pltpu.make_async_copy(
                bufs[c % nbuf], xs_hbm.at[pl.ds(base + c * ch, ch)],
                osem[c % nbuf])
            cp.start()
            return cp

        in_cp = {}
        out_cp = {}
        for c in range(min(look, nch)):
            in_cp[c] = start_in(c)
        for c in range(nch):
            nxt = c + look
            if nxt < nch:
                if nxt >= nbuf:
                    out_cp[nxt - nbuf].wait()
                in_cp[nxt] = start_in(nxt)
            in_cp[c].wait()
            out_cp[c] = start_out(c)
        for c in range(max(0, nch - nbuf), nch):
            out_cp[c].wait()

    return moe_sc_gather(sorted_tok, xf)


def _sc_combine(ys, pos0, pos1, t_tokens):
    rows_w = t_tokens // _NW
    ch = 16
    nch = rows_w // ch
    mesh = plsc.VectorSubcoreMesh(core_axis_name="c", subcore_axis_name="s")

    nslot = 3
    @functools.partial(
        pl.kernel,
        out_type=jax.ShapeDtypeStruct((t_tokens, H), jnp.float32),
        mesh=mesh,
        scratch_types=[
            pltpu.VMEM((rows_w,), jnp.int32),
            pltpu.VMEM((rows_w,), jnp.int32),
            *[pltpu.VMEM((ch, H), jnp.float32) for _ in range(2 * nslot)],
            *[pltpu.SemaphoreType.DMA for _ in range(3 * nslot)],
        ],
    )
    def moe_sc_combine(p0_hbm, p1_hbm, ys_hbm, out_hbm, i0_v, i1_v,
                       *bufs_sems):
        bufa = bufs_sems[:nslot]
        bufb = bufs_sems[nslot:2 * nslot]
        sema = bufs_sems[2 * nslot:3 * nslot]
        semb = bufs_sems[3 * nslot:4 * nslot]
        semo = bufs_sems[4 * nslot:]
        wid = lax.axis_index("s") * _NC + lax.axis_index("c")
        base = wid * rows_w
        pltpu.sync_copy(p0_hbm.at[pl.ds(base, rows_w)], i0_v)
        pltpu.sync_copy(p1_hbm.at[pl.ds(base, rows_w)], i1_v)

        def start_in(c):
            s = c % nslot
            cpa = pltpu.make_async_copy(
                ys_hbm.at[i0_v.at[pl.ds(c * ch, ch)]], bufa[s], sema[s])
            cpb = pltpu.make_async_copy(
                ys_hbm.at[i1_v.at[pl.ds(c * ch, ch)]], bufb[s], semb[s])
            cpa.start()
            cpb.start()
            return cpa, cpb

        in_cp = {}
        out_cp = {}
        for c in range(min(1, nch)):
            in_cp[c] = start_in(c)
        for c in range(nch):
            s = c % nslot
            nxt = c + 1
            if nxt < nch:
                if nxt >= nslot:
                    out_cp[nxt - nslot].wait()
                in_cp[nxt] = start_in(nxt)
            in_cp[c][0].wait()
            in_cp[c][1].wait()

            def body(r, _):
                for o in range(H // 16):
                    sl = pl.ds(o * 16, 16)
                    bufa[s][r, sl] = bufa[s][r, sl] + bufb[s][r, sl]
                return 0

            lax.fori_loop(0, ch, body, 0)
            cpo = pltpu.make_async_copy(
                bufa[s], out_hbm.at[pl.ds(base + c * ch, ch)], semo[s])
            cpo.start()
            out_cp[c] = cpo
        for c in range(max(0, nch - nslot), nch):
            out_cp[c].wait()

    return moe_sc_combine(pos0, pos1, ys)


def kernel(x, gate_w, gate_b, w1, b1, w2, b2):
    B, S, _ = x.shape
    T = B * S
    A = K * T
    nb = A // BLK + E
    padded = nb * BLK

    xf = x.reshape(T, H)

    logits = xf @ gate_w + gate_b
    w = jax.nn.softmax(logits, axis=-1)
    tkw, tki = jax.lax.top_k(w, K)
    tkw = tkw / jnp.clip(jnp.sum(tkw, axis=-1, keepdims=True), 1e-6, None)

    e_flat = tki.reshape(-1).astype(jnp.int32)
    w_flat = tkw.reshape(-1)
    tok_flat = (jnp.arange(A, dtype=jnp.int32) // K)

    onehot = (e_flat[:, None] == jnp.arange(E, dtype=jnp.int32)[None, :])
    counts = jnp.sum(onehot.astype(jnp.int32), axis=0)
    seg_len = ((counts + BLK - 1) // BLK) * BLK
    starts = jnp.concatenate([jnp.zeros((1,), jnp.int32),
                              jnp.cumsum(seg_len)[:-1].astype(jnp.int32)])
    rank = jnp.cumsum(onehot.astype(jnp.int32), axis=0) - 1
    rank_own = jnp.take_along_axis(rank, e_flat[:, None], axis=1)[:, 0]
    pos = starts[e_flat] + rank_own

    ends = starts + seg_len
    block_start = jnp.arange(nb, dtype=jnp.int32) * BLK
    block_expert = jnp.sum(
        (block_start[:, None] >= ends[None, :]).astype(jnp.int32),
        axis=1).astype(jnp.int32)

    sorted_tok = jnp.zeros((padded,), jnp.int32).at[pos].set(tok_flat)
    sorted_cw = jnp.zeros((padded,), jnp.float32).at[pos].set(w_flat)
    xs = _sc_gather(sorted_tok, xf, padded, T)
    cw2 = jnp.broadcast_to(sorted_cw[:, None], (padded, 128))

    ys = _gmm(block_expert, xs, cw2, w1, b1, w2, b2, nb)

    pos2 = pos.reshape(T, K)
    out = _sc_combine(ys, pos2[:, 0], pos2[:, 1], T)
    return out.reshape(B, S, H)
